# baseline (speedup 1.0000x reference)
import functools

import jax
import jax.numpy as jnp
from jax import lax
from jax.experimental import pallas as pl
from jax.experimental.pallas import tpu as pltpu
from jax.experimental.pallas import tpu_sc as plsc

LARGE_POSITION_VALUE = 1000000
MAX_SEQ_LEN = 512
EMB = 64
B, L = 4096, 200
NW = 32
RPW = B // NW
GROUPS = RPW // 16
SPLIT = 128
NBUF = 4
AHEAD = 2
UNROLL = 4


def _body(vco_hbm, pe_hbm, out_hbm, vco_v, rbuf, *sems):
    gsem = sems[:NBUF]
    osem = sems[NBUF:]
    cid = lax.axis_index("c")
    sid = lax.axis_index("s")
    wid = sid * 2 + cid
    row0 = wid * RPW

    pltpu.sync_copy(vco_hbm.at[pl.ds(row0, RPW)], vco_v)

    lanes = lax.broadcasted_iota(jnp.int32, (16,), 0)
    for g in range(GROUPS):
        rows = lanes + g * 16

        def min_step(i, m):
            for k in range(UNROLL):
                col = plsc.load_gather(vco_v, [rows, jnp.full((16,), i * UNROLL + k, jnp.int32)])
                m = jnp.minimum(m, jnp.where(col > 0, col, LARGE_POSITION_VALUE))
            return m

        m = lax.fori_loop(
            0, L // UNROLL, min_step,
            jnp.full((16,), LARGE_POSITION_VALUE, jnp.int32),
        )

        def ord_step(i, carry):
            for k in range(UNROLL):
                idx = jnp.full((16,), i * UNROLL + k, jnp.int32)
                col = plsc.load_gather(vco_v, [rows, idx])
                o = jnp.minimum(jnp.maximum(col - m, 0), jnp.int32(MAX_SEQ_LEN - 1))
                plsc.store_scatter(vco_v, [rows, idx], o)
            return carry

        lax.fori_loop(0, L // UNROLL, ord_step, jnp.int32(0))

    def gather(r, s):
        pltpu.async_copy(
            pe_hbm.at[vco_v.at[r, pl.ds(0, SPLIT)]],
            rbuf.at[s].at[pl.ds(0, SPLIT)],
            gsem[s],
        )
        pltpu.async_copy(
            pe_hbm.at[vco_v.at[r, pl.ds(SPLIT, L - SPLIT)]],
            rbuf.at[s].at[pl.ds(SPLIT, L - SPLIT)],
            gsem[s],
        )

    def wait_gather(s):
        pltpu.make_async_copy(
            pe_hbm.at[vco_v.at[0, pl.ds(0, SPLIT)]],
            rbuf.at[s].at[pl.ds(0, SPLIT)],
            gsem[s],
        ).wait()
        pltpu.make_async_copy(
            pe_hbm.at[vco_v.at[0, pl.ds(SPLIT, L - SPLIT)]],
            rbuf.at[s].at[pl.ds(SPLIT, L - SPLIT)],
            gsem[s],
        ).wait()

    def out_start(r, s):
        pltpu.make_async_copy(rbuf.at[s], out_hbm.at[row0 + r], osem[s]).start()

    def wait_out(r, s):
        pltpu.make_async_copy(rbuf.at[s], out_hbm.at[row0 + r], osem[s]).wait()

    for r in range(AHEAD):
        gather(r, r % NBUF)

    def row_step(i, carry):
        for s in range(NBUF):
            r = i * NBUF + s
            wait_gather(s)
            out_start(r, s)
            nxt = (s + AHEAD) % NBUF

            @pl.when(r >= AHEAD)
            def _():
                wait_out(r - AHEAD, nxt)

            @pl.when(r + AHEAD < RPW)
            def _():
                gather(r + AHEAD, nxt)

        return carry

    lax.fori_loop(0, RPW // NBUF, row_step, jnp.int32(0))
    for r in range(RPW - AHEAD, RPW):
        wait_out(r, r % NBUF)


def kernel(visit_concept_orders, pe):
    mesh = plsc.VectorSubcoreMesh(core_axis_name="c", subcore_axis_name="s")
    run = functools.partial(
        pl.kernel,
        mesh=mesh,
        compiler_params=pltpu.CompilerParams(
            needs_layout_passes=False, use_tc_tiling_on_sc=False
        ),
        out_type=jax.ShapeDtypeStruct((B, L, EMB), jnp.float32),
        scratch_types=[
            pltpu.VMEM((RPW, L), jnp.int32),
            pltpu.VMEM((NBUF, L, EMB), jnp.float32),
        ] + [pltpu.SemaphoreType.DMA] * (2 * NBUF),
    )(_body)
    return run(visit_concept_orders, pe)


# speedup vs baseline: 1.1487x; 1.1487x over previous
import functools

import jax
import jax.numpy as jnp
from jax import lax
from jax.experimental import pallas as pl
from jax.experimental.pallas import tpu as pltpu
from jax.experimental.pallas import tpu_sc as plsc

LARGE_POSITION_VALUE = 1000000
MAX_SEQ_LEN = 512
EMB = 64
B, L = 4096, 200
NW = 32
RPW = B // NW
GROUPS = RPW // 16
NSLAB = 4
UNROLL = 4
TC_N = B // 128
TR_N = EMB // 8


def _body(vco_hbm, pet_hbm, out_hbm, vco_v, ordt_v, pet_v, slab_v, *sems):
    cid = lax.axis_index("c")
    sid = lax.axis_index("s")
    wid = sid * 2 + cid
    row0 = wid * RPW

    pltpu.sync_copy(vco_hbm.at[pl.ds(row0, RPW)], vco_v)
    pltpu.sync_copy(pet_hbm, pet_v)

    lanes = lax.broadcasted_iota(jnp.int32, (16,), 0)
    for g in range(GROUPS):
        rows = lanes + g * 16

        def min_step(i, m):
            for k in range(UNROLL):
                col = plsc.load_gather(vco_v, [rows, jnp.full((16,), i * UNROLL + k, jnp.int32)])
                m = jnp.minimum(m, jnp.where(col > 0, col, LARGE_POSITION_VALUE))
            return m

        m = lax.fori_loop(
            0, L // UNROLL, min_step,
            jnp.full((16,), LARGE_POSITION_VALUE, jnp.int32),
        )

        def ord_step(i, carry):
            for k in range(UNROLL):
                l = i * UNROLL + k
                col = plsc.load_gather(vco_v, [rows, jnp.full((16,), l, jnp.int32)])
                o = jnp.minimum(jnp.maximum(col - m, 0), jnp.int32(MAX_SEQ_LEN - 1))
                plsc.store_scatter(ordt_v, [l * 128 + rows], o)
            return carry

        lax.fori_loop(0, L // UNROLL, ord_step, jnp.int32(0))

    def outs(l, s):
        for tr in range(TR_N):
            yield pltpu.make_async_copy(
                slab_v.at[s].at[pl.ds(tr * 1024, 1024)],
                out_hbm.at[l, tr, wid],
                sems[s],
            )

    def build(l, s):
        for g in range(GROUPS):
            base = ordt_v[pl.ds(l * 128 + g * 16, 16)]
            for e in range(EMB):
                v = plsc.load_gather(pet_v, [base + e * MAX_SEQ_LEN])
                slab_v.at[s][pl.ds(e * 128 + g * 16, 16)] = v

    def l_step(i, carry):
        for s in range(NSLAB):
            l = i * NSLAB + s

            @pl.when(l >= NSLAB)
            def _():
                for cp in outs(l - NSLAB, s):
                    cp.wait()

            build(l, s)
            for cp in outs(l, s):
                cp.start()

        return carry

    lax.fori_loop(0, L // NSLAB, l_step, jnp.int32(0))
    for l in range(L - NSLAB, L):
        for cp in outs(l, l % NSLAB):
            cp.wait()


def kernel(visit_concept_orders, pe):
    mesh = plsc.VectorSubcoreMesh(core_axis_name="c", subcore_axis_name="s")
    run = functools.partial(
        pl.kernel,
        mesh=mesh,
        compiler_params=pltpu.CompilerParams(
            needs_layout_passes=False, use_tc_tiling_on_sc=False
        ),
        out_type=jax.ShapeDtypeStruct((L, TR_N, TC_N, 1024), jnp.float32),
        scratch_types=[
            pltpu.VMEM((RPW, L), jnp.int32),
            pltpu.VMEM((RPW * L,), jnp.int32),
            pltpu.VMEM((EMB * MAX_SEQ_LEN,), jnp.float32),
            pltpu.VMEM((NSLAB, EMB * 128), jnp.float32),
        ] + [pltpu.SemaphoreType.DMA] * NSLAB,
    )(_body)
    pet = jnp.swapaxes(pe, 0, 1).reshape(EMB * MAX_SEQ_LEN)
    z = run(visit_concept_orders, pet)
    y = z.reshape(L, TR_N, TC_N, 8, 128).transpose(2, 4, 0, 1, 3)
    return y.reshape(B, L, EMB)


# speedup vs baseline: 2.0802x; 1.8109x over previous
import functools

import jax
import jax.numpy as jnp
from jax import lax
from jax.experimental import pallas as pl
from jax.experimental.pallas import tpu as pltpu
from jax.experimental.pallas import tpu_sc as plsc

LARGE_POSITION_VALUE = 1000000
MAX_SEQ_LEN = 512
EMB = 64
B, L = 4096, 200
NW = 32
RPW = B // NW
GROUPS = RPW // 16
NSLAB = 4
UNROLL = 4
TC_N = B // 128
TR_N = EMB // 8


def _body(vco_hbm, pet_hbm, out_hbm, vco_v, ordt_v, pet_v, slab_v, *sems):
    cid = lax.axis_index("c")
    sid = lax.axis_index("s")
    wid = sid * 2 + cid
    row0 = wid * RPW

    pltpu.sync_copy(vco_hbm.at[pl.ds(row0, RPW)], vco_v)
    pltpu.sync_copy(pet_hbm, pet_v)

    lanes = lax.broadcasted_iota(jnp.int32, (16,), 0)
    for g in range(GROUPS):
        rows = lanes + g * 16

        def min_step(i, m):
            for k in range(UNROLL):
                col = plsc.load_gather(vco_v, [rows, jnp.full((16,), i * UNROLL + k, jnp.int32)])
                m = jnp.minimum(m, jnp.where(col > 0, col, LARGE_POSITION_VALUE))
            return m

        m = lax.fori_loop(
            0, L // UNROLL, min_step,
            jnp.full((16,), LARGE_POSITION_VALUE, jnp.int32),
        )

        def ord_step(i, carry):
            for k in range(UNROLL):
                l = i * UNROLL + k
                col = plsc.load_gather(vco_v, [rows, jnp.full((16,), l, jnp.int32)])
                o = jnp.minimum(jnp.maximum(col - m, 0), jnp.int32(MAX_SEQ_LEN - 1))
                plsc.store_scatter(ordt_v, [l * 128 + rows], o)
            return carry

        lax.fori_loop(0, L // UNROLL, ord_step, jnp.int32(0))

    def outs(l, s):
        for tr in range(TR_N):
            yield pltpu.make_async_copy(
                slab_v.at[s].at[pl.ds(tr * 1024, 1024)],
                out_hbm.at[l, tr, wid],
                sems[s],
            )

    GQ = 8

    def build(l, s):
        for g in range(GROUPS):
            base = ordt_v[pl.ds(l * 128 + g * 16, 16)]
            for eq in range(EMB // GQ):
                vs = [
                    plsc.load_gather(pet_v, [base + (eq * GQ + j) * MAX_SEQ_LEN])
                    for j in range(GQ)
                ]
                for j, v in enumerate(vs):
                    slab_v.at[s][pl.ds((eq * GQ + j) * 128 + g * 16, 16)] = v

    def l_step(i, carry):
        for s in range(NSLAB):
            l = i * NSLAB + s

            @pl.when(l >= NSLAB)
            def _():
                for cp in outs(l - NSLAB, s):
                    cp.wait()

            build(l, s)
            for cp in outs(l, s):
                cp.start()

        return carry

    lax.fori_loop(0, L // NSLAB, l_step, jnp.int32(0))
    for l in range(L - NSLAB, L):
        for cp in outs(l, l % NSLAB):
            cp.wait()


def kernel(visit_concept_orders, pe):
    mesh = plsc.VectorSubcoreMesh(core_axis_name="c", subcore_axis_name="s")
    run = functools.partial(
        pl.kernel,
        mesh=mesh,
        compiler_params=pltpu.CompilerParams(
            needs_layout_passes=False, use_tc_tiling_on_sc=False
        ),
        out_type=jax.ShapeDtypeStruct((L, TR_N, TC_N, 1024), jnp.float32),
        scratch_types=[
            pltpu.VMEM((RPW, L), jnp.int32),
            pltpu.VMEM((RPW * L,), jnp.int32),
            pltpu.VMEM((EMB * MAX_SEQ_LEN,), jnp.float32),
            pltpu.VMEM((NSLAB, EMB * 128), jnp.float32),
        ] + [pltpu.SemaphoreType.DMA] * NSLAB,
    )(_body)
    pet = jnp.swapaxes(pe, 0, 1).reshape(EMB * MAX_SEQ_LEN)
    z = run(visit_concept_orders, pet)
    y = z.reshape(L, TR_N, TC_N, 8, 128).transpose(2, 4, 0, 1, 3)
    return y.reshape(B, L, EMB)
